# baseline (speedup 1.0000x reference)
import functools

import numpy as np
import jax
import jax.numpy as jnp
from jax import lax
from jax.experimental import pallas as pl
from jax.experimental.pallas import tpu as pltpu
from jax.experimental.pallas import tpu_sc as plsc

_VOCAB = 1000000
_D = 64
_B = 32
_S = 2048

_NC = 2
_NS = 16
_NW = _NC * _NS
_CHUNK = _S // _NW
_L = 16


def _pe_blocks():
    pos = np.arange(_S, dtype=np.float64)[:, None]
    i = np.arange(0, _D, 2, dtype=np.float64)
    angle = pos / (10000.0 ** (2.0 * i / _D))
    pe = np.zeros((_S, _D), dtype=np.float64)
    pe[:, 0::2] = np.sin(angle)
    pe[:, 1::2] = np.cos(angle)
    blocks = pe.reshape(_NW, _CHUNK, _D).transpose(0, 2, 1)
    return jnp.asarray(np.ascontiguousarray(blocks).reshape(-1),
                       dtype=jnp.float32)


def _build_kernel():
    mesh = plsc.VectorSubcoreMesh(
        core_axis_name="c", subcore_axis_name="s",
        num_cores=_NC, num_subcores=_NS,
    )

    @functools.partial(
        pl.kernel,
        mesh=mesh,
        compiler_params=pltpu.CompilerParams(needs_layout_passes=False),
        out_type=jax.ShapeDtypeStruct((_B, _S, _D), jnp.float32),
        scratch_types=[
            pltpu.VMEM((_B, _CHUNK), jnp.int32),
            pltpu.VMEM((_B, _CHUNK), jnp.int32),
            pltpu.VMEM((_B, _CHUNK), jnp.int32),
            pltpu.VMEM((_CHUNK * _D,), jnp.float32),
            pltpu.VMEM((2, _CHUNK, 2 * _D), jnp.float32),
            pltpu.VMEM((_CHUNK, _D), jnp.float32),
            pltpu.SemaphoreType.DMA((2,)),
            pltpu.SemaphoreType.DMA,
        ],
    )
    def emb_kernel(x_h, t2_h, peb_h, out_h,
                   idx_v, idx2_v, p64_v, pe_v, buf, obuf, sems, isem):
        wid = lax.axis_index("s") * _NC + lax.axis_index("c")
        base = wid * _CHUNK

        pltpu.sync_copy(peb_h.at[pl.ds(wid * (_CHUNK * _D), _CHUNK * _D)],
                        pe_v)
        idx_copies = [
            pltpu.async_copy(
                x_h.at[pl.ds(b * _S + base, _CHUNK)], idx_v.at[b], isem)
            for b in range(_B)
        ]
        for cp in idx_copies:
            cp.wait()

        def split_ids(i, _):
            b = i // (_CHUNK // _L)
            g = i % (_CHUNK // _L)
            sl = pl.ds(g * _L, _L)
            t = idx_v[b, sl]
            idx2_v[b, sl] = lax.shift_right_logical(t, 1)
            p64_v[b, sl] = lax.shift_left(
                jnp.bitwise_and(t, jnp.int32(1)), 6)
            return 0
        lax.fori_loop(0, _B * (_CHUNK // _L), split_ids, 0)

        copies = [None, None]
        copies[0] = pltpu.async_copy(
            t2_h.at[idx2_v.at[0]], buf.at[0], sems.at[0])

        jl = [lax.iota(jnp.int32, _L) + g * _L
              for g in range(_CHUNK // _L)]

        for b in range(_B):
            slot = b % 2
            copies[slot].wait()
            if b + 1 < _B:
                nxt = (b + 1) % 2
                copies[nxt] = pltpu.async_copy(
                    t2_h.at[idx2_v.at[b + 1]], buf.at[nxt], sems.at[nxt])

            for g in range(_CHUNK // _L):
                pvec = p64_v[b, pl.ds(g * _L, _L)]

                def body(c, _, g=g, pvec=pvec):
                    col = pvec + c
                    vals = plsc.load_gather(buf.at[slot], [jl[g], col])
                    pe16 = pe_v[pl.ds(c * _CHUNK + g * _L, _L)]
                    cvec = jnp.full((_L,), 0, jnp.int32) + c
                    plsc.store_scatter(obuf, [jl[g], cvec], vals + pe16)
                    return 0
                lax.fori_loop(0, _D, body, 0)

            pltpu.sync_copy(obuf, out_h.at[b, pl.ds(base, _CHUNK), :])

    return emb_kernel


_EMB_KERNEL = None


def kernel(x, table):
    global _EMB_KERNEL
    if _EMB_KERNEL is None:
        _EMB_KERNEL = _build_kernel()
    t2 = jnp.reshape(table, (_VOCAB // 2, 2 * _D))
    return _EMB_KERNEL(jnp.reshape(x, (_B * _S,)), t2, _pe_blocks())
